# baseline (speedup 1.0000x reference)
import jax
import jax.numpy as jnp
from jax.experimental import pallas as pl

N = 10000
NUM_LAYERS = 3


def _layer(x, pos, src, dst, region_mask, W1, b1, W2, b2, Wlin, blin, Wroot):
    rel = pos[src] - pos[dst]
    h = jax.nn.relu(rel @ W1 + b1)
    w = jax.nn.sigmoid(h @ W2 + b2)
    xt = x @ Wlin + blin
    msg = xt[src] * w * region_mask
    agg = jax.ops.segment_sum(msg, dst, num_segments=x.shape[0])
    deg = jax.ops.segment_sum(region_mask[:, 0], dst, num_segments=x.shape[0])
    deg = jnp.maximum(deg, 1.0)[:, None]
    return jax.nn.relu(agg / deg + x @ Wroot)


def kernel(x, edge_index, pos, node_region,
           W1_0, b1_0, W2_0, b2_0, Wlin_0, blin_0, Wroot_0,
           W1_1, b1_1, W2_1, b2_1, Wlin_1, blin_1, Wroot_1,
           W1_2, b1_2, W2_2, b2_2, Wlin_2, blin_2, Wroot_2):
    p = [
        (W1_0, b1_0, W2_0, b2_0, Wlin_0, blin_0, Wroot_0),
        (W1_1, b1_1, W2_1, b2_1, Wlin_1, blin_1, Wroot_1),
        (W1_2, b1_2, W2_2, b2_2, Wlin_2, blin_2, Wroot_2),
    ]
    src = edge_index[0]
    dst = edge_index[1]
    region_mask = (node_region[src] == node_region[dst]).astype(jnp.float32)[:, None]
    h = x
    for i in range(NUM_LAYERS):
        h = _layer(h, pos, src, dst, region_mask, *p[i])
    return h


# speedup vs baseline: 7.2538x; 7.2538x over previous
import functools

import jax
import jax.numpy as jnp
from jax import lax
from jax.experimental import pallas as pl
from jax.experimental.pallas import tpu as pltpu
from jax.experimental.pallas import tpu_sc as plsc

N = 10000
E = 320000
D = 128
NC = 2
NS = 16
NW = NC * NS
EPT = E // NW
K = 80
NCH = EPT // K
RPT = N // NS

_MESH = plsc.VectorSubcoreMesh(core_axis_name="c", subcore_axis_name="s")
_SC_PARAMS = pltpu.CompilerParams(needs_layout_passes=False)


def _zero_vec16(ref, nvec):
    z = jnp.zeros((16,), jnp.float32)

    def body(i, _):
        ref[pl.ds(i * 16, 16)] = z
        return 0

    lax.fori_loop(0, nvec, body, 0)


def _pre_body(posx_h, posy_h, reg_h, src_h, dst_h,
              relx_h, rely_h, mask_h, degp_h,
              posx_v, posy_v, reg_v, srcb, dstb,
              relxb, relyb, maskb, zb, deg_sh, sem):
    c = lax.axis_index("c")
    s = lax.axis_index("s")
    wid = c * NS + s

    pltpu.sync_copy(posx_h, posx_v)
    pltpu.sync_copy(posy_h, posy_v)
    pltpu.sync_copy(reg_h, reg_v)
    pltpu.sync_copy(src_h.at[wid], srcb)
    pltpu.sync_copy(dst_h.at[wid], dstb)

    _zero_vec16(zb, 63)

    @pl.when(s < 10)
    def _():
        pltpu.sync_copy(zb.at[pl.ds(0, 1000)], deg_sh.at[pl.ds(s * 1000, 1000)])

    plsc.subcore_barrier()

    def chunk(j, _):
        for v in range(K // 16):
            sl = pl.ds(v * 16, 16)
            si = srcb[j, sl]
            di = dstb[j, sl]
            pxs = plsc.load_gather(posx_v, [si])
            pxd = plsc.load_gather(posx_v, [di])
            pys = plsc.load_gather(posy_v, [si])
            pyd = plsc.load_gather(posy_v, [di])
            rs = plsc.load_gather(reg_v, [si])
            rd = plsc.load_gather(reg_v, [di])
            relxb[j, sl] = pxs - pxd
            relyb[j, sl] = pys - pyd
            maskb[j, sl] = jnp.where(rs == rd, 1.0, 0.0).astype(jnp.float32)
        pltpu.sync_copy(maskb.at[j], deg_sh.at[dstb.at[j]], add=True)
        return 0

    lax.fori_loop(0, NCH, chunk, 0)

    pltpu.sync_copy(relxb, relx_h.at[wid])
    pltpu.sync_copy(relyb, rely_h.at[wid])
    pltpu.sync_copy(maskb, mask_h.at[wid])

    plsc.subcore_barrier()

    @pl.when(s < 10)
    def _():
        pltpu.sync_copy(deg_sh.at[pl.ds(s * 1000, 1000)], zb.at[pl.ds(0, 1000)])
        pltpu.sync_copy(zb.at[pl.ds(0, 1000)],
                        degp_h.at[pl.ds(c * N + s * 1000, 1000)])


def _sc_preprocess(posx, posy, region, src3, dst3):
    f32 = jnp.float32
    return pl.kernel(
        _pre_body,
        out_type=(
            jax.ShapeDtypeStruct((NW, NCH, K), f32),
            jax.ShapeDtypeStruct((NW, NCH, K), f32),
            jax.ShapeDtypeStruct((NW, NCH, K), f32),
            jax.ShapeDtypeStruct((NC * N,), f32),
        ),
        mesh=_MESH,
        compiler_params=_SC_PARAMS,
        scratch_types=[
            pltpu.VMEM((N,), f32),
            pltpu.VMEM((N,), f32),
            pltpu.VMEM((N,), jnp.int32),
            pltpu.VMEM((NCH, K), jnp.int32),
            pltpu.VMEM((NCH, K), jnp.int32),
            pltpu.VMEM((NCH, K), f32),
            pltpu.VMEM((NCH, K), f32),
            pltpu.VMEM((NCH, K), f32),
            pltpu.VMEM((1008,), f32),
            pltpu.VMEM_SHARED((N,), f32),
            pltpu.SemaphoreType.DMA,
        ],
    )(posx, posy, region, src3, dst3)


def _spmm_body(xt_h, we_h, src_h, dst_h, aggp_h,
               srcb, dstb, web, rows, zb2, agg_sh, sem):
    c = lax.axis_index("c")
    s = lax.axis_index("s")
    wid = c * NS + s

    z = jnp.zeros((16,), jnp.float32)

    def zrow(r, _):
        for k in range(8):
            zb2[r, pl.ds(k * 16, 16)] = z
        return 0

    lax.fori_loop(0, 125, zrow, 0)

    def zagg(k, _):
        pltpu.sync_copy(zb2, agg_sh.at[pl.ds(s * RPT + k * 125, 125)])
        return 0

    lax.fori_loop(0, 5, zagg, 0)

    plsc.subcore_barrier()

    def superchunk(sc, _):
        pltpu.sync_copy(src_h.at[wid, sc], srcb)
        pltpu.sync_copy(dst_h.at[wid, sc], dstb)
        pltpu.sync_copy(we_h.at[wid, sc], web)

        def chunk(j, _1):
            pltpu.async_copy(xt_h.at[srcb.at[j]], rows, sem).wait()

            def scale(g, _2):
                wvec = web[j, pl.ds(g * 16, 16)]
                for i in range(16):
                    wb = jnp.full((16,), wvec[i], jnp.float32)
                    r = g * 16 + i
                    for k in range(8):
                        sl = pl.ds(k * 16, 16)
                        rows[r, sl] = rows[r, sl] * wb
                return 0

            lax.fori_loop(0, K // 16, scale, 0)
            pltpu.sync_copy(rows, agg_sh.at[dstb.at[j]], add=True)
            return 0

        lax.fori_loop(0, NCH // 5, chunk, 0)
        return 0

    lax.fori_loop(0, 5, superchunk, 0)

    plsc.subcore_barrier()

    @pl.when(s < 10)
    def _():
        def rdbk(k, _):
            sl = pl.ds(s * 1000 + k * 40, 40)
            pltpu.sync_copy(agg_sh.at[sl], zb2.at[pl.ds(0, 40)])
            pltpu.sync_copy(zb2.at[pl.ds(0, 40)], aggp_h.at[c, sl])
            return 0

        lax.fori_loop(0, 25, rdbk, 0)


def _sc_spmm(xt, we4, src4, dst4):
    f32 = jnp.float32
    return pl.kernel(
        _spmm_body,
        out_type=jax.ShapeDtypeStruct((NC, N, D), f32),
        mesh=_MESH,
        compiler_params=_SC_PARAMS,
        scratch_types=[
            pltpu.VMEM((NCH // 5, K), jnp.int32),
            pltpu.VMEM((NCH // 5, K), jnp.int32),
            pltpu.VMEM((NCH // 5, K), f32),
            pltpu.VMEM((K, D), f32),
            pltpu.VMEM((125, D), f32),
            pltpu.VMEM_SHARED((N, D), f32),
            pltpu.SemaphoreType.DMA,
        ],
    )(xt, we4, src4, dst4)


def _matmul_body(x_ref, w_ref, b_ref, o_ref):
    o_ref[...] = jnp.dot(x_ref[...], w_ref[...],
                         preferred_element_type=jnp.float32) + b_ref[...]


def _tc_xtransform(x, wcat, bcat):
    bn = 1000
    return pl.pallas_call(
        _matmul_body,
        grid=(N // bn,),
        in_specs=[
            pl.BlockSpec((bn, D), lambda i: (i, 0)),
            pl.BlockSpec((D, 2 * D), lambda i: (0, 0)),
            pl.BlockSpec((1, 2 * D), lambda i: (0, 0)),
        ],
        out_specs=pl.BlockSpec((bn, 2 * D), lambda i: (i, 0)),
        out_shape=jax.ShapeDtypeStruct((N, 2 * D), jnp.float32),
    )(x, wcat, bcat)


def _edgew_body(rel8_ref, m_ref, w18_ref, w2_ref, b2_ref, o_ref):
    h = jnp.maximum(jnp.dot(rel8_ref[...], w18_ref[...],
                            preferred_element_type=jnp.float32), 0.0)
    sct = jnp.dot(h, w2_ref[...], preferred_element_type=jnp.float32) + b2_ref[...]
    w = 1.0 / (1.0 + jnp.exp(-sct))
    o_ref[...] = w * m_ref[...]


def _tc_edge_weights(rel8, mask1, w18, w2, b2):
    be = 2000
    return pl.pallas_call(
        _edgew_body,
        grid=(E // be,),
        in_specs=[
            pl.BlockSpec((be, 8), lambda i: (i, 0)),
            pl.BlockSpec((be, 1), lambda i: (i, 0)),
            pl.BlockSpec((8, D), lambda i: (0, 0)),
            pl.BlockSpec((D, 1), lambda i: (0, 0)),
            pl.BlockSpec((1, 1), lambda i: (0, 0)),
        ],
        out_specs=pl.BlockSpec((be, 1), lambda i: (i, 0)),
        out_shape=jax.ShapeDtypeStruct((E, 1), jnp.float32),
    )(rel8, mask1, w18, w2, b2)


def _combine_body(a0_ref, a1_ref, xr_ref, d0_ref, d1_ref, o_ref):
    deg = jnp.maximum(d0_ref[...] + d1_ref[...], 1.0)
    o_ref[...] = jnp.maximum((a0_ref[...] + a1_ref[...]) / deg + xr_ref[...], 0.0)


def _tc_combine(a0, a1, xr, d0, d1):
    bn = 1000
    return pl.pallas_call(
        _combine_body,
        grid=(N // bn,),
        in_specs=[
            pl.BlockSpec((bn, D), lambda i: (i, 0)),
            pl.BlockSpec((bn, D), lambda i: (i, 0)),
            pl.BlockSpec((bn, D), lambda i: (i, 0)),
            pl.BlockSpec((bn, 1), lambda i: (i, 0)),
            pl.BlockSpec((bn, 1), lambda i: (i, 0)),
        ],
        out_specs=pl.BlockSpec((bn, D), lambda i: (i, 0)),
        out_shape=jax.ShapeDtypeStruct((N, D), jnp.float32),
    )(a0, a1, xr, d0, d1)


def kernel(x, edge_index, pos, node_region,
           W1_0, b1_0, W2_0, b2_0, Wlin_0, blin_0, Wroot_0,
           W1_1, b1_1, W2_1, b2_1, Wlin_1, blin_1, Wroot_1,
           W1_2, b1_2, W2_2, b2_2, Wlin_2, blin_2, Wroot_2):
    layers = [
        (W1_0, b1_0, W2_0, b2_0, Wlin_0, blin_0, Wroot_0),
        (W1_1, b1_1, W2_1, b2_1, Wlin_1, blin_1, Wroot_1),
        (W1_2, b1_2, W2_2, b2_2, Wlin_2, blin_2, Wroot_2),
    ]
    src3 = edge_index[0].reshape(NW, NCH, K)
    dst3 = edge_index[1].reshape(NW, NCH, K)
    src4 = src3.reshape(NW, 5, NCH // 5, K)
    dst4 = dst3.reshape(NW, 5, NCH // 5, K)
    posx = pos[:, 0]
    posy = pos[:, 1]

    relx3, rely3, mask3, degp = _sc_preprocess(posx, posy, node_region, src3, dst3)

    relx = relx3.reshape(E)
    rely = rely3.reshape(E)
    mask1 = mask3.reshape(E, 1)
    rel8 = jnp.concatenate(
        [relx[:, None], rely[:, None], jnp.ones((E, 1), jnp.float32),
         jnp.zeros((E, 5), jnp.float32)], axis=1)
    degp2 = degp.reshape(NC, N)
    d0 = degp2[0][:, None]
    d1 = degp2[1][:, None]

    h = x
    for (W1, b1, W2, b2, Wlin, blin, Wroot) in layers:
        wcat = jnp.concatenate([Wlin, Wroot], axis=1)
        bcat = jnp.concatenate([blin, jnp.zeros((D,), jnp.float32)])[None, :]
        xtr = _tc_xtransform(h, wcat, bcat)
        xt = xtr[:, :D]
        xr = xtr[:, D:]

        w18 = jnp.concatenate([W1, b1[None, :], jnp.zeros((5, D), jnp.float32)], axis=0)
        we = _tc_edge_weights(rel8, mask1, w18, W2, b2[None, :])
        we4 = we.reshape(NW, 5, NCH // 5, K)

        aggp = _sc_spmm(xt, we4, src4, dst4)
        h = _tc_combine(aggp[0], aggp[1], xr, d0, d1)
    return h


# speedup vs baseline: 8.0769x; 1.1135x over previous
import functools

import jax
import jax.numpy as jnp
from jax import lax
from jax.experimental import pallas as pl
from jax.experimental.pallas import tpu as pltpu
from jax.experimental.pallas import tpu_sc as plsc

N = 10000
E = 320000
D = 128
NC = 2
NS = 16
NW = NC * NS
EPT = E // NW
K = 80
NCH = EPT // K
RPT = N // NS

_MESH = plsc.VectorSubcoreMesh(core_axis_name="c", subcore_axis_name="s")
_SC_PARAMS = pltpu.CompilerParams(needs_layout_passes=False)


def _zero_vec16(ref, nvec):
    z = jnp.zeros((16,), jnp.float32)

    def body(i, _):
        ref[pl.ds(i * 16, 16)] = z
        return 0

    lax.fori_loop(0, nvec, body, 0)


def _pre_body(posx_h, posy_h, reg_h, src_h, dst_h,
              relx_h, rely_h, mask_h, degp_h,
              posx_v, posy_v, reg_v, srcb, dstb,
              relxb, relyb, maskb, zb, deg_sh, sem):
    c = lax.axis_index("c")
    s = lax.axis_index("s")
    wid = c * NS + s

    pltpu.sync_copy(posx_h, posx_v)
    pltpu.sync_copy(posy_h, posy_v)
    pltpu.sync_copy(reg_h, reg_v)
    pltpu.sync_copy(src_h.at[wid], srcb)
    pltpu.sync_copy(dst_h.at[wid], dstb)

    _zero_vec16(zb, 63)

    @pl.when(s < 10)
    def _():
        pltpu.sync_copy(zb.at[pl.ds(0, 1000)], deg_sh.at[pl.ds(s * 1000, 1000)])

    plsc.subcore_barrier()

    def chunk(j, _):
        for v in range(K // 16):
            sl = pl.ds(v * 16, 16)
            si = srcb[j, sl]
            di = dstb[j, sl]
            pxs = plsc.load_gather(posx_v, [si])
            pxd = plsc.load_gather(posx_v, [di])
            pys = plsc.load_gather(posy_v, [si])
            pyd = plsc.load_gather(posy_v, [di])
            rs = plsc.load_gather(reg_v, [si])
            rd = plsc.load_gather(reg_v, [di])
            relxb[j, sl] = pxs - pxd
            relyb[j, sl] = pys - pyd
            maskb[j, sl] = jnp.where(rs == rd, 1.0, 0.0).astype(jnp.float32)
        pltpu.sync_copy(maskb.at[j], deg_sh.at[dstb.at[j]], add=True)
        return 0

    lax.fori_loop(0, NCH, chunk, 0)

    pltpu.sync_copy(relxb, relx_h.at[wid])
    pltpu.sync_copy(relyb, rely_h.at[wid])
    pltpu.sync_copy(maskb, mask_h.at[wid])

    plsc.subcore_barrier()

    @pl.when(s < 10)
    def _():
        pltpu.sync_copy(deg_sh.at[pl.ds(s * 1000, 1000)], zb.at[pl.ds(0, 1000)])
        pltpu.sync_copy(zb.at[pl.ds(0, 1000)],
                        degp_h.at[pl.ds(c * N + s * 1000, 1000)])


def _sc_preprocess(posx, posy, region, src3, dst3):
    f32 = jnp.float32
    return pl.kernel(
        _pre_body,
        out_type=(
            jax.ShapeDtypeStruct((NW, NCH, K), f32),
            jax.ShapeDtypeStruct((NW, NCH, K), f32),
            jax.ShapeDtypeStruct((NW, NCH, K), f32),
            jax.ShapeDtypeStruct((NC * N,), f32),
        ),
        mesh=_MESH,
        compiler_params=_SC_PARAMS,
        scratch_types=[
            pltpu.VMEM((N,), f32),
            pltpu.VMEM((N,), f32),
            pltpu.VMEM((N,), jnp.int32),
            pltpu.VMEM((NCH, K), jnp.int32),
            pltpu.VMEM((NCH, K), jnp.int32),
            pltpu.VMEM((NCH, K), f32),
            pltpu.VMEM((NCH, K), f32),
            pltpu.VMEM((NCH, K), f32),
            pltpu.VMEM((1008,), f32),
            pltpu.VMEM_SHARED((N,), f32),
            pltpu.SemaphoreType.DMA,
        ],
    )(posx, posy, region, src3, dst3)


def _spmm_body(xt_h, we_h, src_h, dst_h, aggp_h,
               srcb, dstb, web, rows, zb2, agg_sh, gsa, gsb, ssa, ssb):
    c = lax.axis_index("c")
    s = lax.axis_index("s")
    wid = c * NS + s
    rowsb = zb2.at[pl.ds(0, K)]

    z = jnp.zeros((16,), jnp.float32)

    def zrow(r, _):
        for k in range(8):
            zb2[r, pl.ds(k * 16, 16)] = z
        return 0

    lax.fori_loop(0, 125, zrow, 0)

    def zagg(k, _):
        pltpu.sync_copy(zb2, agg_sh.at[pl.ds(s * RPT + k * 125, 125)])
        return 0

    lax.fori_loop(0, 5, zagg, 0)

    plsc.subcore_barrier()

    def scale(buf, we_row, g, _2):
        wvec = web[we_row, pl.ds(g * 16, 16)]
        for i in range(16):
            wb = jnp.full((16,), wvec[i], jnp.float32)
            r = g * 16 + i
            for k in range(8):
                sl = pl.ds(k * 16, 16)
                buf[r, sl] = buf[r, sl] * wb
        return 0

    def superchunk(sc, _):
        pltpu.sync_copy(src_h.at[wid, sc], srcb)
        pltpu.sync_copy(dst_h.at[wid, sc], dstb)
        pltpu.sync_copy(we_h.at[wid, sc], web)

        pltpu.async_copy(xt_h.at[srcb.at[0]], rows, gsa)

        def pair(i, _1):
            ja = 2 * i
            jb = ja + 1
            pltpu.make_async_copy(xt_h.at[srcb.at[ja]], rows, gsa).wait()
            pltpu.async_copy(xt_h.at[srcb.at[jb]], rowsb, gsb)
            lax.fori_loop(0, K // 16, functools.partial(scale, rows, ja), 0)
            da = pltpu.async_copy(rows, agg_sh.at[dstb.at[ja]], ssa, add=True)
            pltpu.make_async_copy(xt_h.at[srcb.at[jb]], rowsb, gsb).wait()
            da.wait()
            pltpu.async_copy(xt_h.at[srcb.at[ja + 2]], rows, gsa)
            lax.fori_loop(0, K // 16, functools.partial(scale, rowsb, jb), 0)
            pltpu.async_copy(rowsb, agg_sh.at[dstb.at[jb]], ssb, add=True).wait()
            return 0

        lax.fori_loop(0, 12, pair, 0)

        jt = 24
        pltpu.make_async_copy(xt_h.at[srcb.at[jt]], rows, gsa).wait()
        lax.fori_loop(0, K // 16, functools.partial(scale, rows, jt), 0)
        pltpu.async_copy(rows, agg_sh.at[dstb.at[jt]], ssa, add=True).wait()
        return 0

    lax.fori_loop(0, 5, superchunk, 0)

    plsc.subcore_barrier()

    @pl.when(s < 10)
    def _():
        def rdbk(k, _):
            sl = pl.ds(s * 1000 + k * 40, 40)
            pltpu.sync_copy(agg_sh.at[sl], zb2.at[pl.ds(0, 40)])
            pltpu.sync_copy(zb2.at[pl.ds(0, 40)], aggp_h.at[c, sl])
            return 0

        lax.fori_loop(0, 25, rdbk, 0)


def _sc_spmm(xt, we4, src4, dst4):
    f32 = jnp.float32
    return pl.kernel(
        _spmm_body,
        out_type=jax.ShapeDtypeStruct((NC, N, D), f32),
        mesh=_MESH,
        compiler_params=_SC_PARAMS,
        scratch_types=[
            pltpu.VMEM((NCH // 5, K), jnp.int32),
            pltpu.VMEM((NCH // 5, K), jnp.int32),
            pltpu.VMEM((NCH // 5, K), f32),
            pltpu.VMEM((K, D), f32),
            pltpu.VMEM((125, D), f32),
            pltpu.VMEM_SHARED((N, D), f32),
            pltpu.SemaphoreType.DMA,
            pltpu.SemaphoreType.DMA,
            pltpu.SemaphoreType.DMA,
            pltpu.SemaphoreType.DMA,
        ],
    )(xt, we4, src4, dst4)


def _matmul_body(x_ref, w_ref, b_ref, o_ref):
    o_ref[...] = jnp.dot(x_ref[...], w_ref[...],
                         preferred_element_type=jnp.float32) + b_ref[...]


def _tc_xtransform(x, wcat, bcat):
    bn = 1000
    return pl.pallas_call(
        _matmul_body,
        grid=(N // bn,),
        in_specs=[
            pl.BlockSpec((bn, D), lambda i: (i, 0)),
            pl.BlockSpec((D, 2 * D), lambda i: (0, 0)),
            pl.BlockSpec((1, 2 * D), lambda i: (0, 0)),
        ],
        out_specs=pl.BlockSpec((bn, 2 * D), lambda i: (i, 0)),
        out_shape=jax.ShapeDtypeStruct((N, 2 * D), jnp.float32),
    )(x, wcat, bcat)


def _edgew_body(rel8_ref, m_ref, w18_ref, w2_ref, b2_ref, o_ref):
    h = jnp.maximum(jnp.dot(rel8_ref[...], w18_ref[...],
                            preferred_element_type=jnp.float32), 0.0)
    sct = jnp.dot(h, w2_ref[...], preferred_element_type=jnp.float32) + b2_ref[...]
    w = 1.0 / (1.0 + jnp.exp(-sct))
    o_ref[...] = w * m_ref[...]


def _tc_edge_weights(rel8, mask1, w18, w2, b2):
    be = 2000
    return pl.pallas_call(
        _edgew_body,
        grid=(E // be,),
        in_specs=[
            pl.BlockSpec((be, 8), lambda i: (i, 0)),
            pl.BlockSpec((be, 1), lambda i: (i, 0)),
            pl.BlockSpec((8, D), lambda i: (0, 0)),
            pl.BlockSpec((D, 1), lambda i: (0, 0)),
            pl.BlockSpec((1, 1), lambda i: (0, 0)),
        ],
        out_specs=pl.BlockSpec((be, 1), lambda i: (i, 0)),
        out_shape=jax.ShapeDtypeStruct((E, 1), jnp.float32),
    )(rel8, mask1, w18, w2, b2)


def _combine_body(a0_ref, a1_ref, xr_ref, d0_ref, d1_ref, o_ref):
    deg = jnp.maximum(d0_ref[...] + d1_ref[...], 1.0)
    o_ref[...] = jnp.maximum((a0_ref[...] + a1_ref[...]) / deg + xr_ref[...], 0.0)


def _tc_combine(a0, a1, xr, d0, d1):
    bn = 1000
    return pl.pallas_call(
        _combine_body,
        grid=(N // bn,),
        in_specs=[
            pl.BlockSpec((bn, D), lambda i: (i, 0)),
            pl.BlockSpec((bn, D), lambda i: (i, 0)),
            pl.BlockSpec((bn, D), lambda i: (i, 0)),
            pl.BlockSpec((bn, 1), lambda i: (i, 0)),
            pl.BlockSpec((bn, 1), lambda i: (i, 0)),
        ],
        out_specs=pl.BlockSpec((bn, D), lambda i: (i, 0)),
        out_shape=jax.ShapeDtypeStruct((N, D), jnp.float32),
    )(a0, a1, xr, d0, d1)


def kernel(x, edge_index, pos, node_region,
           W1_0, b1_0, W2_0, b2_0, Wlin_0, blin_0, Wroot_0,
           W1_1, b1_1, W2_1, b2_1, Wlin_1, blin_1, Wroot_1,
           W1_2, b1_2, W2_2, b2_2, Wlin_2, blin_2, Wroot_2):
    layers = [
        (W1_0, b1_0, W2_0, b2_0, Wlin_0, blin_0, Wroot_0),
        (W1_1, b1_1, W2_1, b2_1, Wlin_1, blin_1, Wroot_1),
        (W1_2, b1_2, W2_2, b2_2, Wlin_2, blin_2, Wroot_2),
    ]
    src3 = edge_index[0].reshape(NW, NCH, K)
    dst3 = edge_index[1].reshape(NW, NCH, K)
    src4 = src3.reshape(NW, 5, NCH // 5, K)
    dst4 = dst3.reshape(NW, 5, NCH // 5, K)
    posx = pos[:, 0]
    posy = pos[:, 1]

    relx3, rely3, mask3, degp = _sc_preprocess(posx, posy, node_region, src3, dst3)

    relx = relx3.reshape(E)
    rely = rely3.reshape(E)
    mask1 = mask3.reshape(E, 1)
    rel8 = jnp.concatenate(
        [relx[:, None], rely[:, None], jnp.ones((E, 1), jnp.float32),
         jnp.zeros((E, 5), jnp.float32)], axis=1)
    degp2 = degp.reshape(NC, N)
    d0 = degp2[0][:, None]
    d1 = degp2[1][:, None]

    h = x
    for (W1, b1, W2, b2, Wlin, blin, Wroot) in layers:
        wcat = jnp.concatenate([Wlin, Wroot], axis=1)
        bcat = jnp.concatenate([blin, jnp.zeros((D,), jnp.float32)])[None, :]
        xtr = _tc_xtransform(h, wcat, bcat)
        xt = xtr[:, :D]
        xr = xtr[:, D:]

        w18 = jnp.concatenate([W1, b1[None, :], jnp.zeros((5, D), jnp.float32)], axis=0)
        we = _tc_edge_weights(rel8, mask1, w18, W2, b2[None, :])
        we4 = we.reshape(NW, 5, NCH // 5, K)

        aggp = _sc_spmm(xt, we4, src4, dst4)
        h = _tc_combine(aggp[0], aggp[1], xr, d0, d1)
    return h


# speedup vs baseline: 8.1917x; 1.0142x over previous
import functools

import jax
import jax.numpy as jnp
from jax import lax
from jax.experimental import pallas as pl
from jax.experimental.pallas import tpu as pltpu
from jax.experimental.pallas import tpu_sc as plsc

N = 10000
E = 320000
D = 128
NC = 2
NS = 16
NW = NC * NS
EPT = E // NW
K = 80
NCH = EPT // K
RPT = N // NS

_MESH = plsc.VectorSubcoreMesh(core_axis_name="c", subcore_axis_name="s")
_SC_PARAMS = pltpu.CompilerParams(needs_layout_passes=False)


def _zero_vec16(ref, nvec):
    z = jnp.zeros((16,), jnp.float32)

    def body(i, _):
        ref[pl.ds(i * 16, 16)] = z
        return 0

    lax.fori_loop(0, nvec, body, 0)


def _pre_body(posx_h, posy_h, reg_h, src_h, dst_h,
              srcC_h, dstC_h, relxC_h, relyC_h, maskC_h, nch_h, degp_h,
              posx_v, posy_v, reg_v, srcb, dstb, maskb,
              srcC, dstC, relxC, relyC, maskC, nchb, zb, deg_sh, sem):
    c = lax.axis_index("c")
    s = lax.axis_index("s")
    wid = c * NS + s

    pltpu.sync_copy(posx_h, posx_v)
    pltpu.sync_copy(posy_h, posy_v)
    pltpu.sync_copy(reg_h, reg_v)

    _zero_vec16(zb, 63)

    zi = jnp.zeros((16,), jnp.int32)
    zf = jnp.zeros((16,), jnp.float32)

    def zcomp(u, _):
        sl = pl.ds(u * 16, 16)
        srcC[0, sl] = zi
        dstC[0, sl] = zi
        relxC[0, sl] = zf
        relyC[0, sl] = zf
        maskC[0, sl] = zf
        return 0

    lax.fori_loop(0, EPT // 16, zcomp, 0)

    @pl.when(s < 10)
    def _():
        pltpu.sync_copy(zb.at[pl.ds(0, 1000)], deg_sh.at[pl.ds(s * 1000, 1000)])

    plsc.subcore_barrier()

    ones16 = jnp.ones((16,), jnp.float32)

    def superchunk(sc, off0):
        pltpu.sync_copy(src_h.at[wid, sc], srcb)
        pltpu.sync_copy(dst_h.at[wid, sc], dstb)

        def chunk(j, off):
            for v in range(K // 16):
                sl = pl.ds(v * 16, 16)
                si = srcb[j, sl]
                di = dstb[j, sl]
                pxs = plsc.load_gather(posx_v, [si])
                pxd = plsc.load_gather(posx_v, [di])
                pys = plsc.load_gather(posy_v, [si])
                pyd = plsc.load_gather(posy_v, [di])
                rs = plsc.load_gather(reg_v, [si])
                rd = plsc.load_gather(reg_v, [di])
                mb = rs == rd
                maskb[j, sl] = jnp.where(mb, 1.0, 0.0).astype(jnp.float32)
                osl = pl.ds(off, 16)
                plsc.store_compressed(srcC.at[0, osl], si, mask=mb)
                plsc.store_compressed(dstC.at[0, osl], di, mask=mb)
                plsc.store_compressed(relxC.at[0, osl], pxs - pxd, mask=mb)
                plsc.store_compressed(relyC.at[0, osl], pys - pyd, mask=mb)
                plsc.store_compressed(maskC.at[0, osl], ones16, mask=mb)
                off = off + plsc.all_reduce_population_count(mb)[0]
            pltpu.sync_copy(maskb.at[j], deg_sh.at[dstb.at[j]], add=True)
            return off

        return lax.fori_loop(0, NCH // 5, chunk, off0)

    cnt = lax.fori_loop(0, 5, superchunk, jnp.int32(0))

    nchb[0, pl.ds(0, 16)] = jnp.full((16,), (cnt + K - 1) // K, jnp.int32)

    pltpu.sync_copy(srcC, srcC_h.at[wid])
    pltpu.sync_copy(dstC, dstC_h.at[wid])
    pltpu.sync_copy(relxC, relxC_h.at[wid])
    pltpu.sync_copy(relyC, relyC_h.at[wid])
    pltpu.sync_copy(maskC, maskC_h.at[wid])
    pltpu.sync_copy(nchb, nch_h.at[wid])

    plsc.subcore_barrier()

    @pl.when(s < 10)
    def _():
        pltpu.sync_copy(deg_sh.at[pl.ds(s * 1000, 1000)], zb.at[pl.ds(0, 1000)])
        pltpu.sync_copy(zb.at[pl.ds(0, 1000)],
                        degp_h.at[pl.ds(c * N + s * 1000, 1000)])


def _sc_preprocess(posx, posy, region, src3, dst3):
    f32 = jnp.float32
    i32 = jnp.int32
    return pl.kernel(
        _pre_body,
        out_type=(
            jax.ShapeDtypeStruct((NW, 1, EPT), i32),
            jax.ShapeDtypeStruct((NW, 1, EPT), i32),
            jax.ShapeDtypeStruct((NW, 1, EPT), f32),
            jax.ShapeDtypeStruct((NW, 1, EPT), f32),
            jax.ShapeDtypeStruct((NW, 1, EPT), f32),
            jax.ShapeDtypeStruct((NW, 1, 16), i32),
            jax.ShapeDtypeStruct((NC * N,), f32),
        ),
        mesh=_MESH,
        compiler_params=_SC_PARAMS,
        scratch_types=[
            pltpu.VMEM((N,), f32),
            pltpu.VMEM((N,), f32),
            pltpu.VMEM((N,), i32),
            pltpu.VMEM((NCH // 5, K), i32),
            pltpu.VMEM((NCH // 5, K), i32),
            pltpu.VMEM((NCH // 5, K), f32),
            pltpu.VMEM((1, EPT), i32),
            pltpu.VMEM((1, EPT), i32),
            pltpu.VMEM((1, EPT), f32),
            pltpu.VMEM((1, EPT), f32),
            pltpu.VMEM((1, EPT), f32),
            pltpu.VMEM((1, 16), i32),
            pltpu.VMEM((1008,), f32),
            pltpu.VMEM_SHARED((N,), f32),
            pltpu.SemaphoreType.DMA,
        ],
    )(posx, posy, region, src3, dst3)


def _spmm_body(xt_h, we_h, src_h, dst_h, nch_h, aggp_h,
               srcb, dstb, web, rows, zb2, nchb, agg_sh, sem):
    c = lax.axis_index("c")
    s = lax.axis_index("s")
    wid = c * NS + s

    pltpu.sync_copy(nch_h.at[wid], nchb)
    nch = nchb[0, pl.ds(0, 16)][0]

    z = jnp.zeros((16,), jnp.float32)

    def zrow(r, _):
        for k in range(8):
            zb2[r, pl.ds(k * 16, 16)] = z
        return 0

    lax.fori_loop(0, 125, zrow, 0)

    def zagg(k, _):
        pltpu.sync_copy(zb2, agg_sh.at[pl.ds(s * RPT + k * 125, 125)])
        return 0

    lax.fori_loop(0, 5, zagg, 0)

    plsc.subcore_barrier()

    def scale(we_row, g, _2):
        wvec = web[we_row, pl.ds(g * 16, 16)]
        for i in range(16):
            wb = jnp.full((16,), wvec[i], jnp.float32)
            r = g * 16 + i
            for k in range(8):
                sl = pl.ds(k * 16, 16)
                rows[r, sl] = rows[r, sl] * wb
        return 0

    def superchunk(sc, _):
        base = sc * 25

        @pl.when(base < nch)
        def _():
            pltpu.sync_copy(src_h.at[wid, sc], srcb)
            pltpu.sync_copy(dst_h.at[wid, sc], dstb)
            pltpu.sync_copy(we_h.at[wid, sc], web)
            jmax = jnp.minimum(25, nch - base)

            def chunk(j, _1):
                pltpu.async_copy(xt_h.at[srcb.at[j]], rows, sem).wait()
                lax.fori_loop(0, K // 16, functools.partial(scale, j), 0)
                pltpu.sync_copy(rows, agg_sh.at[dstb.at[j]], add=True)
                return 0

            lax.fori_loop(0, jmax, chunk, 0)

        return 0

    lax.fori_loop(0, 5, superchunk, 0)

    plsc.subcore_barrier()

    @pl.when(s < 10)
    def _():
        def rdbk(k, _):
            sl = pl.ds(s * 1000 + k * 40, 40)
            pltpu.sync_copy(agg_sh.at[sl], zb2.at[pl.ds(0, 40)])
            pltpu.sync_copy(zb2.at[pl.ds(0, 40)], aggp_h.at[c, sl])
            return 0

        lax.fori_loop(0, 25, rdbk, 0)


def _sc_spmm(xt, we4, src4, dst4, nch3):
    f32 = jnp.float32
    return pl.kernel(
        _spmm_body,
        out_type=jax.ShapeDtypeStruct((NC, N, D), f32),
        mesh=_MESH,
        compiler_params=_SC_PARAMS,
        scratch_types=[
            pltpu.VMEM((NCH // 5, K), jnp.int32),
            pltpu.VMEM((NCH // 5, K), jnp.int32),
            pltpu.VMEM((NCH // 5, K), f32),
            pltpu.VMEM((K, D), f32),
            pltpu.VMEM((125, D), f32),
            pltpu.VMEM((1, 16), jnp.int32),
            pltpu.VMEM_SHARED((N, D), f32),
            pltpu.SemaphoreType.DMA,
        ],
    )(xt, we4, src4, dst4, nch3)


def _matmul_body(x_ref, w_ref, b_ref, o_ref):
    o_ref[...] = jnp.dot(x_ref[...], w_ref[...],
                         preferred_element_type=jnp.float32) + b_ref[...]


def _tc_xtransform(x, wcat, bcat):
    bn = 1000
    return pl.pallas_call(
        _matmul_body,
        grid=(N // bn,),
        in_specs=[
            pl.BlockSpec((bn, D), lambda i: (i, 0)),
            pl.BlockSpec((D, 2 * D), lambda i: (0, 0)),
            pl.BlockSpec((1, 2 * D), lambda i: (0, 0)),
        ],
        out_specs=pl.BlockSpec((bn, 2 * D), lambda i: (i, 0)),
        out_shape=jax.ShapeDtypeStruct((N, 2 * D), jnp.float32),
    )(x, wcat, bcat)


def _edgew_body(rel8_ref, m_ref, w18_ref, w2_ref, b2_ref, o_ref):
    h = jnp.maximum(jnp.dot(rel8_ref[...], w18_ref[...],
                            preferred_element_type=jnp.float32), 0.0)
    sct = jnp.dot(h, w2_ref[...], preferred_element_type=jnp.float32) + b2_ref[...]
    w = 1.0 / (1.0 + jnp.exp(-sct))
    o_ref[...] = w * m_ref[...]


def _tc_edge_weights(rel8, mask1, w18, w2, b2):
    be = 2000
    return pl.pallas_call(
        _edgew_body,
        grid=(E // be,),
        in_specs=[
            pl.BlockSpec((be, 8), lambda i: (i, 0)),
            pl.BlockSpec((be, 1), lambda i: (i, 0)),
            pl.BlockSpec((8, D), lambda i: (0, 0)),
            pl.BlockSpec((D, 1), lambda i: (0, 0)),
            pl.BlockSpec((1, 1), lambda i: (0, 0)),
        ],
        out_specs=pl.BlockSpec((be, 1), lambda i: (i, 0)),
        out_shape=jax.ShapeDtypeStruct((E, 1), jnp.float32),
    )(rel8, mask1, w18, w2, b2)


def _combine_body(a0_ref, a1_ref, xr_ref, d0_ref, d1_ref, o_ref):
    deg = jnp.maximum(d0_ref[...] + d1_ref[...], 1.0)
    o_ref[...] = jnp.maximum((a0_ref[...] + a1_ref[...]) / deg + xr_ref[...], 0.0)


def _tc_combine(a0, a1, xr, d0, d1):
    bn = 1000
    return pl.pallas_call(
        _combine_body,
        grid=(N // bn,),
        in_specs=[
            pl.BlockSpec((bn, D), lambda i: (i, 0)),
            pl.BlockSpec((bn, D), lambda i: (i, 0)),
            pl.BlockSpec((bn, D), lambda i: (i, 0)),
            pl.BlockSpec((bn, 1), lambda i: (i, 0)),
            pl.BlockSpec((bn, 1), lambda i: (i, 0)),
        ],
        out_specs=pl.BlockSpec((bn, D), lambda i: (i, 0)),
        out_shape=jax.ShapeDtypeStruct((N, D), jnp.float32),
    )(a0, a1, xr, d0, d1)


def kernel(x, edge_index, pos, node_region,
           W1_0, b1_0, W2_0, b2_0, Wlin_0, blin_0, Wroot_0,
           W1_1, b1_1, W2_1, b2_1, Wlin_1, blin_1, Wroot_1,
           W1_2, b1_2, W2_2, b2_2, Wlin_2, blin_2, Wroot_2):
    layers = [
        (W1_0, b1_0, W2_0, b2_0, Wlin_0, blin_0, Wroot_0),
        (W1_1, b1_1, W2_1, b2_1, Wlin_1, blin_1, Wroot_1),
        (W1_2, b1_2, W2_2, b2_2, Wlin_2, blin_2, Wroot_2),
    ]
    srcE = edge_index[0].reshape(NW, 5, NCH // 5, K)
    dstE = edge_index[1].reshape(NW, 5, NCH // 5, K)
    posx = pos[:, 0]
    posy = pos[:, 1]

    srcC, dstC, relxC, relyC, maskC, nchO, degp = _sc_preprocess(
        posx, posy, node_region, srcE, dstE)

    src4 = srcC.reshape(NW, 5, NCH // 5, K)
    dst4 = dstC.reshape(NW, 5, NCH // 5, K)
    nch3 = nchO
    relx = relxC.reshape(E)
    rely = relyC.reshape(E)
    mask1 = maskC.reshape(E, 1)
    rel8 = jnp.concatenate(
        [relx[:, None], rely[:, None], jnp.ones((E, 1), jnp.float32),
         jnp.zeros((E, 5), jnp.float32)], axis=1)
    degp2 = degp.reshape(NC, N)
    d0 = degp2[0][:, None]
    d1 = degp2[1][:, None]

    h = x
    for (W1, b1, W2, b2, Wlin, blin, Wroot) in layers:
        wcat = jnp.concatenate([Wlin, Wroot], axis=1)
        bcat = jnp.concatenate([blin, jnp.zeros((D,), jnp.float32)])[None, :]
        xtr = _tc_xtransform(h, wcat, bcat)
        xt = xtr[:, :D]
        xr = xtr[:, D:]

        w18 = jnp.concatenate([W1, b1[None, :], jnp.zeros((5, D), jnp.float32)], axis=0)
        we = _tc_edge_weights(rel8, mask1, w18, W2, b2[None, :])
        we4 = we.reshape(NW, 5, NCH // 5, K)

        aggp = _sc_spmm(xt, we4, src4, dst4, nch3)
        h = _tc_combine(aggp[0], aggp[1], xr, d0, d1)
    return h


# speedup vs baseline: 28.1649x; 3.4382x over previous
import functools

import jax
import jax.numpy as jnp
from jax import lax
from jax.experimental import pallas as pl
from jax.experimental.pallas import tpu as pltpu
from jax.experimental.pallas import tpu_sc as plsc

N = 10000
E = 320000
D = 128
NC = 2
NS = 16
NW = NC * NS
EPT = E // NW
KP = 80
NCHP = EPT // KP
EPTC = 10240
SCE = 2048
K = 64
RPT = N // NS

_MESH = plsc.VectorSubcoreMesh(core_axis_name="c", subcore_axis_name="s")
_SC_PARAMS = pltpu.CompilerParams(needs_layout_passes=False)


def _zero_vec16(ref, nvec):
    z = jnp.zeros((16,), jnp.float32)

    def body(i, _):
        ref[pl.ds(i * 16, 16)] = z
        return 0

    lax.fori_loop(0, nvec, body, 0)


def _pre_body(posx_h, posy_h, reg_h, src_h, dst_h,
              srcC_h, dstC_h, relxC_h, relyC_h, maskC_h, nch_h, degp_h,
              posx_v, posy_v, reg_v, srcb, dstb, maskb,
              srcC, dstC, relxC, relyC, maskC, nchb, zb, deg_sh, sem):
    c = lax.axis_index("c")
    s = lax.axis_index("s")
    wid = c * NS + s

    pltpu.sync_copy(posx_h, posx_v)
    pltpu.sync_copy(posy_h, posy_v)
    pltpu.sync_copy(reg_h, reg_v)

    _zero_vec16(zb, 63)

    zi = jnp.zeros((16,), jnp.int32)
    zf = jnp.zeros((16,), jnp.float32)

    def zcomp(u, _):
        sl = pl.ds(u * 16, 16)
        srcC[0, sl] = zi
        dstC[0, sl] = zi
        relxC[0, sl] = zf
        relyC[0, sl] = zf
        maskC[0, sl] = zf
        return 0

    lax.fori_loop(0, EPTC // 16, zcomp, 0)

    @pl.when(s < 10)
    def _():
        pltpu.sync_copy(zb.at[pl.ds(0, 1000)], deg_sh.at[pl.ds(s * 1000, 1000)])

    plsc.subcore_barrier()

    ones16 = jnp.ones((16,), jnp.float32)

    def superchunk(sc, off0):
        pltpu.sync_copy(src_h.at[wid, sc], srcb)
        pltpu.sync_copy(dst_h.at[wid, sc], dstb)

        def chunk(j, off):
            for v in range(KP // 16):
                sl = pl.ds(v * 16, 16)
                si = srcb[j, sl]
                di = dstb[j, sl]
                pxs = plsc.load_gather(posx_v, [si])
                pxd = plsc.load_gather(posx_v, [di])
                pys = plsc.load_gather(posy_v, [si])
                pyd = plsc.load_gather(posy_v, [di])
                rs = plsc.load_gather(reg_v, [si])
                rd = plsc.load_gather(reg_v, [di])
                mb = rs == rd
                maskb[j, sl] = jnp.where(mb, 1.0, 0.0).astype(jnp.float32)
                osl = pl.ds(off, 16)
                plsc.store_compressed(srcC.at[0, osl], si, mask=mb)
                plsc.store_compressed(dstC.at[0, osl], di, mask=mb)
                plsc.store_compressed(relxC.at[0, osl], pxs - pxd, mask=mb)
                plsc.store_compressed(relyC.at[0, osl], pys - pyd, mask=mb)
                plsc.store_compressed(maskC.at[0, osl], ones16, mask=mb)
                off = off + plsc.all_reduce_population_count(mb)[0]
            pltpu.sync_copy(maskb.at[j], deg_sh.at[dstb.at[j]], add=True)
            return off

        return lax.fori_loop(0, NCHP // 5, chunk, off0)

    cnt = lax.fori_loop(0, 5, superchunk, jnp.int32(0))

    nchb[0, pl.ds(0, 16)] = jnp.full((16,), (cnt + K - 1) // K, jnp.int32)

    def wrout(sc, _):
        sl = pl.ds(sc * SCE, SCE)
        pltpu.sync_copy(srcC.at[:, sl], srcC_h.at[wid, sc])
        pltpu.sync_copy(dstC.at[:, sl], dstC_h.at[wid, sc])
        pltpu.sync_copy(relxC.at[:, sl], relxC_h.at[wid, sc])
        pltpu.sync_copy(relyC.at[:, sl], relyC_h.at[wid, sc])
        pltpu.sync_copy(maskC.at[:, sl], maskC_h.at[wid, sc])
        return 0

    lax.fori_loop(0, EPTC // SCE, wrout, 0)
    pltpu.sync_copy(nchb, nch_h.at[wid])

    plsc.subcore_barrier()

    @pl.when(s < 10)
    def _():
        pltpu.sync_copy(deg_sh.at[pl.ds(s * 1000, 1000)], zb.at[pl.ds(0, 1000)])
        pltpu.sync_copy(zb.at[pl.ds(0, 1000)],
                        degp_h.at[pl.ds(c * N + s * 1000, 1000)])


def _sc_preprocess(posx, posy, region, src4, dst4):
    f32 = jnp.float32
    i32 = jnp.int32
    nsc = EPTC // SCE
    return pl.kernel(
        _pre_body,
        out_type=(
            jax.ShapeDtypeStruct((NW, nsc, 1, SCE), i32),
            jax.ShapeDtypeStruct((NW, nsc, 1, SCE), i32),
            jax.ShapeDtypeStruct((NW, nsc, 1, SCE), f32),
            jax.ShapeDtypeStruct((NW, nsc, 1, SCE), f32),
            jax.ShapeDtypeStruct((NW, nsc, 1, SCE), f32),
            jax.ShapeDtypeStruct((NW, 1, 16), i32),
            jax.ShapeDtypeStruct((NC * N,), f32),
        ),
        mesh=_MESH,
        compiler_params=_SC_PARAMS,
        scratch_types=[
            pltpu.VMEM((N,), f32),
            pltpu.VMEM((N,), f32),
            pltpu.VMEM((N,), i32),
            pltpu.VMEM((NCHP // 5, KP), i32),
            pltpu.VMEM((NCHP // 5, KP), i32),
            pltpu.VMEM((NCHP // 5, KP), f32),
            pltpu.VMEM((1, EPTC), i32),
            pltpu.VMEM((1, EPTC), i32),
            pltpu.VMEM((1, EPTC), f32),
            pltpu.VMEM((1, EPTC), f32),
            pltpu.VMEM((1, EPTC), f32),
            pltpu.VMEM((1, 16), i32),
            pltpu.VMEM((1008,), f32),
            pltpu.VMEM_SHARED((N,), f32),
            pltpu.SemaphoreType.DMA,
        ],
    )(posx, posy, region, src4, dst4)


def _spmm_body(xt_h, src_h, dst_h, rlx_h, rly_h, msk_h, nch_h,
               w1x_h, w1y_h, b1_h, w2_h, b2_h, aggp_h,
               srcb, dstb, rlxb, rlyb, mskb,
               w1xv, w1yv, b1v, w2v, b2v, nchb, rows, zb2, agg_sh, sem):
    c = lax.axis_index("c")
    s = lax.axis_index("s")
    wid = c * NS + s

    pltpu.sync_copy(nch_h.at[wid], nchb)
    nch = nchb[0, pl.ds(0, 16)][0]

    pltpu.sync_copy(w1x_h, w1xv)
    pltpu.sync_copy(w1y_h, w1yv)
    pltpu.sync_copy(b1_h, b1v)
    pltpu.sync_copy(w2_h, w2v)
    pltpu.sync_copy(b2_h, b2v)

    z = jnp.zeros((16,), jnp.float32)

    def zrow(r, _):
        for k in range(8):
            zb2[r, pl.ds(k * 16, 16)] = z
        return 0

    lax.fori_loop(0, 125, zrow, 0)

    def zagg(k, _):
        pltpu.sync_copy(zb2, agg_sh.at[pl.ds(s * RPT + k * 125, 125)])
        return 0

    lax.fori_loop(0, 5, zagg, 0)

    plsc.subcore_barrier()

    b2s = b2v[pl.ds(0, 16)][0]

    def edge_weight(relx16, rely16, m16):
        def jj_body(jj, acc):
            w1xg = w1xv[pl.ds(jj * 16, 16)]
            w1yg = w1yv[pl.ds(jj * 16, 16)]
            b1g = b1v[pl.ds(jj * 16, 16)]
            w2g = w2v[pl.ds(jj * 16, 16)]
            for i in range(16):
                h = jnp.maximum(relx16 * w1xg[i] + rely16 * w1yg[i] + b1g[i],
                                0.0)
                acc = acc + h * w2g[i]
            return acc

        s16 = lax.fori_loop(0, D // 16, jj_body, jnp.full((16,), b2s, jnp.float32))
        return m16 / (1.0 + jnp.exp(-s16))

    def superchunk(sc, _):
        base = sc * (SCE // K)

        @pl.when(base < nch)
        def _():
            pltpu.sync_copy(src_h.at[wid, sc], srcb)
            pltpu.sync_copy(dst_h.at[wid, sc], dstb)
            pltpu.sync_copy(rlx_h.at[wid, sc], rlxb)
            pltpu.sync_copy(rly_h.at[wid, sc], rlyb)
            pltpu.sync_copy(msk_h.at[wid, sc], mskb)
            jmax = jnp.minimum(SCE // K, nch - base)

            def chunk(j, _1):
                off = j * K
                gd = pltpu.async_copy(
                    xt_h.at[srcb.at[0, pl.ds(off, K)]], rows, sem)
                wes = []
                dsts = []
                for g in range(K // 16):
                    sl = pl.ds(off + g * 16, 16)
                    wes.append(edge_weight(rlxb[0, sl], rlyb[0, sl],
                                           mskb[0, sl]))
                    dsts.append(dstb[0, sl])
                gd.wait()
                for g in range(K // 16):
                    we16 = wes[g]
                    for i in range(16):
                        wb = jnp.full((16,), we16[i], jnp.float32)
                        r = g * 16 + i
                        for k in range(8):
                            sl = pl.ds(k * 16, 16)
                            rows[r, sl] = rows[r, sl] * wb
                for g in range(K // 16):
                    pltpu.sync_copy(rows.at[pl.ds(g * 16, 16)],
                                    agg_sh.at[dsts[g]], add=True)
                return 0

            lax.fori_loop(0, jmax, chunk, 0)

        return 0

    lax.fori_loop(0, EPTC // SCE, superchunk, 0)

    plsc.subcore_barrier()

    @pl.when(s < 10)
    def _():
        def rdbk(k, _):
            sl = pl.ds(s * 1000 + k * 40, 40)
            pltpu.sync_copy(agg_sh.at[sl], zb2.at[pl.ds(0, 40)])
            pltpu.sync_copy(zb2.at[pl.ds(0, 40)], aggp_h.at[c, sl])
            return 0

        lax.fori_loop(0, 25, rdbk, 0)


def _sc_spmm(xt, srcC, dstC, relxC, relyC, maskC, nch3,
             w1x, w1y, b1, w2, b2v):
    f32 = jnp.float32
    i32 = jnp.int32
    return pl.kernel(
        _spmm_body,
        out_type=jax.ShapeDtypeStruct((NC, N, D), f32),
        mesh=_MESH,
        compiler_params=_SC_PARAMS,
        scratch_types=[
            pltpu.VMEM((1, SCE), i32),
            pltpu.VMEM((1, SCE), i32),
            pltpu.VMEM((1, SCE), f32),
            pltpu.VMEM((1, SCE), f32),
            pltpu.VMEM((1, SCE), f32),
            pltpu.VMEM((D,), f32),
            pltpu.VMEM((D,), f32),
            pltpu.VMEM((D,), f32),
            pltpu.VMEM((D,), f32),
            pltpu.VMEM((16,), f32),
            pltpu.VMEM((1, 16), i32),
            pltpu.VMEM((K, D), f32),
            pltpu.VMEM((125, D), f32),
            pltpu.VMEM_SHARED((N, D), f32),
            pltpu.SemaphoreType.DMA,
        ],
    )(xt, srcC, dstC, relxC, relyC, maskC, nch3, w1x, w1y, b1, w2, b2v)


def _matmul_body(x_ref, w_ref, b_ref, o_ref):
    o_ref[...] = jnp.dot(x_ref[...], w_ref[...],
                         preferred_element_type=jnp.float32) + b_ref[...]


def _tc_xtransform(x, wcat, bcat):
    bn = 1000
    return pl.pallas_call(
        _matmul_body,
        grid=(N // bn,),
        in_specs=[
            pl.BlockSpec((bn, D), lambda i: (i, 0)),
            pl.BlockSpec((D, 2 * D), lambda i: (0, 0)),
            pl.BlockSpec((1, 2 * D), lambda i: (0, 0)),
        ],
        out_specs=pl.BlockSpec((bn, 2 * D), lambda i: (i, 0)),
        out_shape=jax.ShapeDtypeStruct((N, 2 * D), jnp.float32),
    )(x, wcat, bcat)


def _combine_body(a_ref, d_ref, xr_ref, o_ref):
    deg = jnp.maximum(d_ref[0] + d_ref[1], 1.0)
    o_ref[...] = jnp.maximum((a_ref[0] + a_ref[1]) / deg + xr_ref[...], 0.0)


def _tc_combine(aggp, degp3, xr):
    bn = 1000
    return pl.pallas_call(
        _combine_body,
        grid=(N // bn,),
        in_specs=[
            pl.BlockSpec((NC, bn, D), lambda i: (0, i, 0)),
            pl.BlockSpec((NC, bn, 1), lambda i: (0, i, 0)),
            pl.BlockSpec((bn, D), lambda i: (i, 0)),
        ],
        out_specs=pl.BlockSpec((bn, D), lambda i: (i, 0)),
        out_shape=jax.ShapeDtypeStruct((N, D), jnp.float32),
    )(aggp, degp3, xr)


def kernel(x, edge_index, pos, node_region,
           W1_0, b1_0, W2_0, b2_0, Wlin_0, blin_0, Wroot_0,
           W1_1, b1_1, W2_1, b2_1, Wlin_1, blin_1, Wroot_1,
           W1_2, b1_2, W2_2, b2_2, Wlin_2, blin_2, Wroot_2):
    layers = [
        (W1_0, b1_0, W2_0, b2_0, Wlin_0, blin_0, Wroot_0),
        (W1_1, b1_1, W2_1, b2_1, Wlin_1, blin_1, Wroot_1),
        (W1_2, b1_2, W2_2, b2_2, Wlin_2, blin_2, Wroot_2),
    ]
    srcE = edge_index[0].reshape(NW, 5, NCHP // 5, KP)
    dstE = edge_index[1].reshape(NW, 5, NCHP // 5, KP)
    posx = pos[:, 0]
    posy = pos[:, 1]

    srcC, dstC, relxC, relyC, maskC, nch3, degp = _sc_preprocess(
        posx, posy, node_region, srcE, dstE)

    degp3 = degp.reshape(NC, N, 1)

    h = x
    for (W1, b1, W2, b2, Wlin, blin, Wroot) in layers:
        wcat = jnp.concatenate([Wlin, Wroot], axis=1)
        bcat = jnp.concatenate([blin, jnp.zeros((D,), jnp.float32)])[None, :]
        xtr = _tc_xtransform(h, wcat, bcat)
        xt = xtr[:, :D]
        xr = xtr[:, D:]

        b2v = jnp.broadcast_to(b2, (16,))
        aggp = _sc_spmm(xt, srcC, dstC, relxC, relyC, maskC, nch3,
                        W1[0], W1[1], b1, W2[:, 0], b2v)
        h = _tc_combine(aggp, degp3, xr)
    return h


# speedup vs baseline: 29.9508x; 1.0634x over previous
import functools

import jax
import jax.numpy as jnp
from jax import lax
from jax.experimental import pallas as pl
from jax.experimental.pallas import tpu as pltpu
from jax.experimental.pallas import tpu_sc as plsc

N = 10000
E = 320000
D = 128
NC = 2
NS = 16
NW = NC * NS
EPT = E // NW
KP = 80
NCHP = EPT // KP
EPTC = 10240
SCE = 2048
K = 64
RPT = N // NS

_MESH = plsc.VectorSubcoreMesh(core_axis_name="c", subcore_axis_name="s")
_SC_PARAMS = pltpu.CompilerParams(needs_layout_passes=False)


def _zero_vec16(ref, nvec):
    z = jnp.zeros((16,), jnp.float32)

    def body(i, _):
        ref[pl.ds(i * 16, 16)] = z
        return 0

    lax.fori_loop(0, nvec, body, 0)


def _pre_body(posx_h, posy_h, reg_h, src_h, dst_h,
              srcC_h, dstC_h, relxC_h, relyC_h, maskC_h, nch_h, degp_h,
              posx_v, posy_v, reg_v, srcb, dstb, maskb,
              srcC, dstC, relxC, relyC, maskC, nchb, zb, deg_sh, sem):
    c = lax.axis_index("c")
    s = lax.axis_index("s")
    wid = c * NS + s

    pltpu.sync_copy(posx_h, posx_v)
    pltpu.sync_copy(posy_h, posy_v)
    pltpu.sync_copy(reg_h, reg_v)

    _zero_vec16(zb, 63)

    zi = jnp.zeros((16,), jnp.int32)
    zf = jnp.zeros((16,), jnp.float32)

    def zcomp(u, _):
        sl = pl.ds(u * 16, 16)
        srcC[0, sl] = zi
        dstC[0, sl] = zi
        relxC[0, sl] = zf
        relyC[0, sl] = zf
        maskC[0, sl] = zf
        return 0

    lax.fori_loop(0, EPTC // 16, zcomp, 0)

    @pl.when(s < 10)
    def _():
        pltpu.sync_copy(zb.at[pl.ds(0, 1000)], deg_sh.at[pl.ds(s * 1000, 1000)])

    plsc.subcore_barrier()

    ones16 = jnp.ones((16,), jnp.float32)

    def superchunk(sc, off0):
        pltpu.sync_copy(src_h.at[wid, sc], srcb)
        pltpu.sync_copy(dst_h.at[wid, sc], dstb)

        def chunk(j, off):
            for v in range(KP // 16):
                sl = pl.ds(v * 16, 16)
                si = srcb[j, sl]
                di = dstb[j, sl]
                pxs = plsc.load_gather(posx_v, [si])
                pxd = plsc.load_gather(posx_v, [di])
                pys = plsc.load_gather(posy_v, [si])
                pyd = plsc.load_gather(posy_v, [di])
                rs = plsc.load_gather(reg_v, [si])
                rd = plsc.load_gather(reg_v, [di])
                mb = rs == rd
                maskb[j, sl] = jnp.where(mb, 1.0, 0.0).astype(jnp.float32)
                osl = pl.ds(off, 16)
                plsc.store_compressed(srcC.at[0, osl], si, mask=mb)
                plsc.store_compressed(dstC.at[0, osl], di, mask=mb)
                plsc.store_compressed(relxC.at[0, osl], pxs - pxd, mask=mb)
                plsc.store_compressed(relyC.at[0, osl], pys - pyd, mask=mb)
                plsc.store_compressed(maskC.at[0, osl], ones16, mask=mb)
                off = off + plsc.all_reduce_population_count(mb)[0]
            pltpu.sync_copy(maskb.at[j], deg_sh.at[dstb.at[j]], add=True)
            return off

        return lax.fori_loop(0, NCHP // 5, chunk, off0)

    cnt = lax.fori_loop(0, 5, superchunk, jnp.int32(0))

    nchb[0, pl.ds(0, 16)] = jnp.full((16,), (cnt + K - 1) // K, jnp.int32)

    def wrout(sc, _):
        sl = pl.ds(sc * SCE, SCE)
        pltpu.sync_copy(srcC.at[:, sl], srcC_h.at[wid, sc])
        pltpu.sync_copy(dstC.at[:, sl], dstC_h.at[wid, sc])
        pltpu.sync_copy(relxC.at[:, sl], relxC_h.at[wid, sc])
        pltpu.sync_copy(relyC.at[:, sl], relyC_h.at[wid, sc])
        pltpu.sync_copy(maskC.at[:, sl], maskC_h.at[wid, sc])
        return 0

    lax.fori_loop(0, EPTC // SCE, wrout, 0)
    pltpu.sync_copy(nchb, nch_h.at[wid])

    plsc.subcore_barrier()

    @pl.when(s < 10)
    def _():
        pltpu.sync_copy(deg_sh.at[pl.ds(s * 1000, 1000)], zb.at[pl.ds(0, 1000)])
        pltpu.sync_copy(zb.at[pl.ds(0, 1000)],
                        degp_h.at[pl.ds(c * N + s * 1000, 1000)])


def _sc_preprocess(posx, posy, region, src4, dst4):
    f32 = jnp.float32
    i32 = jnp.int32
    nsc = EPTC // SCE
    return pl.kernel(
        _pre_body,
        out_type=(
            jax.ShapeDtypeStruct((NW, nsc, 1, SCE), i32),
            jax.ShapeDtypeStruct((NW, nsc, 1, SCE), i32),
            jax.ShapeDtypeStruct((NW, nsc, 1, SCE), f32),
            jax.ShapeDtypeStruct((NW, nsc, 1, SCE), f32),
            jax.ShapeDtypeStruct((NW, nsc, 1, SCE), f32),
            jax.ShapeDtypeStruct((NW, 1, 16), i32),
            jax.ShapeDtypeStruct((NC * N,), f32),
        ),
        mesh=_MESH,
        compiler_params=_SC_PARAMS,
        scratch_types=[
            pltpu.VMEM((N,), f32),
            pltpu.VMEM((N,), f32),
            pltpu.VMEM((N,), i32),
            pltpu.VMEM((NCHP // 5, KP), i32),
            pltpu.VMEM((NCHP // 5, KP), i32),
            pltpu.VMEM((NCHP // 5, KP), f32),
            pltpu.VMEM((1, EPTC), i32),
            pltpu.VMEM((1, EPTC), i32),
            pltpu.VMEM((1, EPTC), f32),
            pltpu.VMEM((1, EPTC), f32),
            pltpu.VMEM((1, EPTC), f32),
            pltpu.VMEM((1, 16), i32),
            pltpu.VMEM((1008,), f32),
            pltpu.VMEM_SHARED((N,), f32),
            pltpu.SemaphoreType.DMA,
        ],
    )(posx, posy, region, src4, dst4)


def _spmm_body(xt_h, src_h, dst_h, rlx_h, rly_h, msk_h, nch_h,
               w1x_h, w1y_h, b1_h, w2_h, b2_h, aggp_h,
               srcb, dstb, rlxb, rlyb, mskb,
               w1xv, w1yv, b1v, w2v, b2v, nchb, rows, zb2, agg_sh, sem):
    c = lax.axis_index("c")
    s = lax.axis_index("s")
    wid = c * NS + s

    pltpu.sync_copy(nch_h.at[wid], nchb)
    nch = nchb[0, pl.ds(0, 16)][0]

    pltpu.sync_copy(w1x_h, w1xv)
    pltpu.sync_copy(w1y_h, w1yv)
    pltpu.sync_copy(b1_h, b1v)
    pltpu.sync_copy(w2_h, w2v)
    pltpu.sync_copy(b2_h, b2v)

    z = jnp.zeros((16,), jnp.float32)

    def zrow(r, _):
        for k in range(8):
            zb2[r, pl.ds(k * 16, 16)] = z
        return 0

    lax.fori_loop(0, 125, zrow, 0)

    def zagg(k, _):
        pltpu.sync_copy(zb2, agg_sh.at[pl.ds(s * RPT + k * 125, 125)])
        return 0

    lax.fori_loop(0, 5, zagg, 0)

    plsc.subcore_barrier()

    b2s = b2v[pl.ds(0, 16)][0]

    def edge_weight(relx16, rely16, m16):
        def jj_body(jj, acc):
            w1xg = w1xv[pl.ds(jj * 16, 16)]
            w1yg = w1yv[pl.ds(jj * 16, 16)]
            b1g = b1v[pl.ds(jj * 16, 16)]
            w2g = w2v[pl.ds(jj * 16, 16)]
            for i in range(16):
                h = jnp.maximum(relx16 * w1xg[i] + rely16 * w1yg[i] + b1g[i],
                                0.0)
                acc = acc + h * w2g[i]
            return acc

        s16 = lax.fori_loop(0, D // 16, jj_body, jnp.full((16,), b2s, jnp.float32))
        return m16 / (1.0 + jnp.exp(-s16))

    def superchunk(sc, _):
        base = sc * (SCE // K)

        @pl.when(base < nch)
        def _():
            pltpu.sync_copy(src_h.at[wid, sc], srcb)
            pltpu.sync_copy(dst_h.at[wid, sc], dstb)
            pltpu.sync_copy(rlx_h.at[wid, sc], rlxb)
            pltpu.sync_copy(rly_h.at[wid, sc], rlyb)
            pltpu.sync_copy(msk_h.at[wid, sc], mskb)
            jmax = jnp.minimum(SCE // K, nch - base)

            def chunk(j, _1):
                off = j * K
                gd = pltpu.async_copy(
                    xt_h.at[srcb.at[0, pl.ds(off, K)]], rows, sem)
                wes = []
                dsts = []
                for g in range(K // 16):
                    sl = pl.ds(off + g * 16, 16)
                    wes.append(edge_weight(rlxb[0, sl], rlyb[0, sl],
                                           mskb[0, sl]))
                    dsts.append(dstb[0, sl])
                gd.wait()
                for g in range(K // 16):
                    we16 = wes[g]
                    for i in range(16):
                        wb = jnp.full((16,), we16[i], jnp.float32)
                        r = g * 16 + i
                        for k in range(8):
                            sl = pl.ds(k * 16, 16)
                            rows[r, sl] = rows[r, sl] * wb
                for g in range(K // 16):
                    pltpu.sync_copy(rows.at[pl.ds(g * 16, 16)],
                                    agg_sh.at[dsts[g]], add=True)
                return 0

            lax.fori_loop(0, jmax, chunk, 0)

        return 0

    lax.fori_loop(0, EPTC // SCE, superchunk, 0)

    plsc.subcore_barrier()

    @pl.when(s < 10)
    def _():
        def rdbk(k, _):
            sl = pl.ds(s * 1000 + k * 120, 120)
            pltpu.sync_copy(agg_sh.at[sl], zb2.at[pl.ds(0, 120)])
            pltpu.sync_copy(zb2.at[pl.ds(0, 120)], aggp_h.at[c, sl])
            return 0

        lax.fori_loop(0, 8, rdbk, 0)
        sl = pl.ds(s * 1000 + 960, 40)
        pltpu.sync_copy(agg_sh.at[sl], zb2.at[pl.ds(0, 40)])
        pltpu.sync_copy(zb2.at[pl.ds(0, 40)], aggp_h.at[c, sl])


def _sc_spmm(xt, srcC, dstC, relxC, relyC, maskC, nch3,
             w1x, w1y, b1, w2, b2v):
    f32 = jnp.float32
    i32 = jnp.int32
    return pl.kernel(
        _spmm_body,
        out_type=jax.ShapeDtypeStruct((NC, N, D), f32),
        mesh=_MESH,
        compiler_params=_SC_PARAMS,
        scratch_types=[
            pltpu.VMEM((1, SCE), i32),
            pltpu.VMEM((1, SCE), i32),
            pltpu.VMEM((1, SCE), f32),
            pltpu.VMEM((1, SCE), f32),
            pltpu.VMEM((1, SCE), f32),
            pltpu.VMEM((D,), f32),
            pltpu.VMEM((D,), f32),
            pltpu.VMEM((D,), f32),
            pltpu.VMEM((D,), f32),
            pltpu.VMEM((16,), f32),
            pltpu.VMEM((1, 16), i32),
            pltpu.VMEM((K, D), f32),
            pltpu.VMEM((125, D), f32),
            pltpu.VMEM_SHARED((N, D), f32),
            pltpu.SemaphoreType.DMA,
        ],
    )(xt, srcC, dstC, relxC, relyC, maskC, nch3, w1x, w1y, b1, w2, b2v)


def _matmul_body(x_ref, w_ref, b_ref, o_ref):
    o_ref[...] = jnp.dot(x_ref[...], w_ref[...],
                         preferred_element_type=jnp.float32) + b_ref[...]


def _tc_xtransform(x, wcat, bcat):
    bn = 1000
    return pl.pallas_call(
        _matmul_body,
        grid=(N // bn,),
        in_specs=[
            pl.BlockSpec((bn, D), lambda i: (i, 0)),
            pl.BlockSpec((D, 2 * D), lambda i: (0, 0)),
            pl.BlockSpec((1, 2 * D), lambda i: (0, 0)),
        ],
        out_specs=pl.BlockSpec((bn, 2 * D), lambda i: (i, 0)),
        out_shape=jax.ShapeDtypeStruct((N, 2 * D), jnp.float32),
    )(x, wcat, bcat)


def _combine_body(a_ref, d_ref, xr_ref, o_ref):
    deg = jnp.maximum(d_ref[0] + d_ref[1], 1.0)
    o_ref[...] = jnp.maximum((a_ref[0] + a_ref[1]) / deg + xr_ref[...], 0.0)


def _tc_combine(aggp, degp3, xr):
    bn = 1000
    return pl.pallas_call(
        _combine_body,
        grid=(N // bn,),
        in_specs=[
            pl.BlockSpec((NC, bn, D), lambda i: (0, i, 0)),
            pl.BlockSpec((NC, bn, 1), lambda i: (0, i, 0)),
            pl.BlockSpec((bn, D), lambda i: (i, 0)),
        ],
        out_specs=pl.BlockSpec((bn, D), lambda i: (i, 0)),
        out_shape=jax.ShapeDtypeStruct((N, D), jnp.float32),
    )(aggp, degp3, xr)


def _combine_mm_body(a_ref, d_ref, xr_ref, w_ref, b_ref, xtr_ref):
    deg = jnp.maximum(d_ref[0] + d_ref[1], 1.0)
    h = jnp.maximum((a_ref[0] + a_ref[1]) / deg + xr_ref[...], 0.0)
    xtr_ref[...] = jnp.dot(h, w_ref[...],
                           preferred_element_type=jnp.float32) + b_ref[...]


def _tc_combine_mm(aggp, degp3, xr, wcat, bcat):
    bn = 1000
    return pl.pallas_call(
        _combine_mm_body,
        grid=(N // bn,),
        in_specs=[
            pl.BlockSpec((NC, bn, D), lambda i: (0, i, 0)),
            pl.BlockSpec((NC, bn, 1), lambda i: (0, i, 0)),
            pl.BlockSpec((bn, D), lambda i: (i, 0)),
            pl.BlockSpec((D, 2 * D), lambda i: (0, 0)),
            pl.BlockSpec((1, 2 * D), lambda i: (0, 0)),
        ],
        out_specs=pl.BlockSpec((bn, 2 * D), lambda i: (i, 0)),
        out_shape=jax.ShapeDtypeStruct((N, 2 * D), jnp.float32),
    )(aggp, degp3, xr, wcat, bcat)


def kernel(x, edge_index, pos, node_region,
           W1_0, b1_0, W2_0, b2_0, Wlin_0, blin_0, Wroot_0,
           W1_1, b1_1, W2_1, b2_1, Wlin_1, blin_1, Wroot_1,
           W1_2, b1_2, W2_2, b2_2, Wlin_2, blin_2, Wroot_2):
    layers = [
        (W1_0, b1_0, W2_0, b2_0, Wlin_0, blin_0, Wroot_0),
        (W1_1, b1_1, W2_1, b2_1, Wlin_1, blin_1, Wroot_1),
        (W1_2, b1_2, W2_2, b2_2, Wlin_2, blin_2, Wroot_2),
    ]
    srcE = edge_index[0].reshape(NW, 5, NCHP // 5, KP)
    dstE = edge_index[1].reshape(NW, 5, NCHP // 5, KP)
    posx = pos[:, 0]
    posy = pos[:, 1]

    srcC, dstC, relxC, relyC, maskC, nch3, degp = _sc_preprocess(
        posx, posy, node_region, srcE, dstE)

    degp3 = degp.reshape(NC, N, 1)

    wcats = []
    bcats = []
    for (W1, b1, W2, b2, Wlin, blin, Wroot) in layers:
        wcats.append(jnp.concatenate([Wlin, Wroot], axis=1))
        bcats.append(
            jnp.concatenate([blin, jnp.zeros((D,), jnp.float32)])[None, :])

    xtr = _tc_xtransform(x, wcats[0], bcats[0])
    for i, (W1, b1, W2, b2, Wlin, blin, Wroot) in enumerate(layers):
        xt = xtr[:, :D]
        xr = xtr[:, D:]
        b2v = jnp.broadcast_to(b2, (16,))
        aggp = _sc_spmm(xt, srcC, dstC, relxC, relyC, maskC, nch3,
                        W1[0], W1[1], b1, W2[:, 0], b2v)
        if i < len(layers) - 1:
            xtr = _tc_combine_mm(aggp, degp3, xr, wcats[i + 1], bcats[i + 1])
        else:
            return _tc_combine(aggp, degp3, xr)
